# baseline (speedup 1.0000x reference)
import functools

import jax
import jax.numpy as jnp
from jax import lax
from jax.experimental import pallas as pl
from jax.experimental.pallas import tpu as pltpu
from jax.experimental.pallas import tpu_sc as plsc

BATCH = 16384
DIM = 64
ROW = 128
L = 16
NC, NS = 2, 16
NW = NC * NS
BPW = BATCH // NW
HALF = BPW // 2
CHUNK = 128
K = DIM // L


def _body(uids_hbm, iids_hbm, utab_hbm, itab_hbm, out_hbm,
          uidx_v, iidx_v, uhidx_v, ihidx_v, ubuf_v, ibuf_v, out_v, sem):
    wid = lax.axis_index("s") * NC + lax.axis_index("c")
    base = wid * BPW

    pltpu.sync_copy(uids_hbm.at[pl.ds(base, BPW)], uidx_v)
    pltpu.sync_copy(iids_hbm.at[pl.ds(base, BPW)], iidx_v)

    for i in range(BPW // L):
        sl = pl.ds(i * L, L)
        uhidx_v[sl] = lax.shift_right_logical(uidx_v[sl], 1)
        ihidx_v[sl] = lax.shift_right_logical(iidx_v[sl], 1)

    rows_iota = lax.iota(jnp.int32, L)

    for h in range(BPW // HALF):
        copies = []
        for j in range(HALF // CHUNK):
            isl = pl.ds(h * HALF + j * CHUNK, CHUNK)
            bsl = pl.ds(j * CHUNK, CHUNK)
            copies.append(pltpu.async_copy(utab_hbm.at[uhidx_v.at[isl]], ubuf_v.at[bsl], sem))
            copies.append(pltpu.async_copy(itab_hbm.at[ihidx_v.at[isl]], ibuf_v.at[bsl], sem))
        for c in copies:
            c.wait()

        def group(g, carry):
            outvec = jnp.zeros((L,), jnp.float32)
            gsl = pl.ds(h * HALF + g * L, L)
            uoffs = (uidx_v[gsl] & 1) * DIM
            ioffs = (iidx_v[gsl] & 1) * DIM
            for r in range(L):
                row = g * L + r
                uoff = uoffs[r]
                ioff = ioffs[r]
                acc = (ubuf_v[row, pl.ds(uoff, L)]
                       * ibuf_v[row, pl.ds(ioff, L)])
                for k in range(1, K):
                    acc = acc + (ubuf_v[row, pl.ds(uoff + k * L, L)]
                                 * ibuf_v[row, pl.ds(ioff + k * L, L)])
                outvec = jnp.where(rows_iota == r, jnp.sum(acc), outvec)
            out_v[pl.ds(h * HALF + g * L, L)] = outvec
            return carry

        lax.fori_loop(0, HALF // L, group, 0)

    pltpu.sync_copy(out_v, out_hbm.at[pl.ds(base, BPW)])


def kernel(user_ids, item_ids, user_table, item_table):
    nu, dim = user_table.shape
    ni = item_table.shape[0]
    utab = user_table.reshape(nu * dim // ROW, ROW)
    itab = item_table.reshape(ni * dim // ROW, ROW)
    mesh = plsc.VectorSubcoreMesh(core_axis_name="c", subcore_axis_name="s")
    run = functools.partial(
        pl.kernel,
        out_type=jax.ShapeDtypeStruct((BATCH,), jnp.float32),
        mesh=mesh,
        compiler_params=pltpu.CompilerParams(needs_layout_passes=False),
        scratch_types=[
            pltpu.VMEM((BPW,), jnp.int32),
            pltpu.VMEM((BPW,), jnp.int32),
            pltpu.VMEM((BPW,), jnp.int32),
            pltpu.VMEM((BPW,), jnp.int32),
            pltpu.VMEM((HALF, ROW), jnp.float32),
            pltpu.VMEM((HALF, ROW), jnp.float32),
            pltpu.VMEM((BPW,), jnp.float32),
            pltpu.SemaphoreType.DMA,
        ],
    )(_body)
    return run(user_ids.astype(jnp.int32), item_ids.astype(jnp.int32),
               utab, itab)


# speedup vs baseline: 3.7597x; 3.7597x over previous
import functools

import jax
import jax.numpy as jnp
from jax import lax
from jax.experimental import pallas as pl
from jax.experimental.pallas import tpu as pltpu
from jax.experimental.pallas import tpu_sc as plsc

BATCH = 16384
DIM = 64
NROWS = 1000000
L = 16
NC, NS = 2, 16
NW = NC * NS
BPW = BATCH // NW
WINC = 512
NWIN_FULL = (NROWS // 128) // 4
WPW = NWIN_FULL // NW
SPECIAL_R0 = NWIN_FULL * WINC
SPECIAL_W = NROWS - SPECIAL_R0
BSHIFT = 14
BMASK = (1 << BSHIFT) - 1


def _drain_one(ph_v, uall_hbm, sem):
    pltpu.make_async_copy(uall_hbm.at[pl.ds(0, DIM)], ph_v.at[0, 0], sem).wait()


def _pass(ids_hbm, tab_hbm, uall_hbm, wid,
          ids_v, slist_v, blist_v, blk_v, st_v, ph_v, semB, semW):
    iota = lax.iota(jnp.int32, L)
    wlo = wid * (WPW * WINC)
    is_last = wid == NW - 1
    whi = jnp.where(is_last, SPECIAL_R0, wlo + WPW * WINC)
    nwin = jnp.where(is_last, WPW + 1, WPW)

    def filt_chunk(i, cnt, qbase):
        ids = ids_v[pl.ds(i * L, L)]
        m = (ids >= wlo) & (ids < whi)
        mi = m.astype(jnp.int32)
        pos = cnt + jnp.cumsum(mi) - mi
        enc = (ids - wlo) * (BMASK + 1) + (qbase + i * L + iota)
        plsc.store_scatter(slist_v, [pos], enc, mask=m)
        return cnt + jnp.sum(mi)

    cnt = jnp.int32(0)
    for q in range(4):
        pltpu.sync_copy(ids_hbm.at[pl.ds(q * 4096, 4096)], ids_v)
        cnt = lax.fori_loop(
            0, 4096 // L,
            functools.partial(filt_chunk, qbase=q * 4096), cnt)

    def proc_block(buf, rlo, rhi, col_base):
        def grp_chunk(i, n):
            encs = slist_v[pl.ds(i * L, L)]
            rloc = lax.shift_right_logical(encs, BSHIFT)
            m = ((iota + i * L) < cnt) & (rloc >= rlo) & (rloc < rhi)
            mi = m.astype(jnp.int32)
            pos = n + jnp.cumsum(mi) - mi
            plsc.store_scatter(blist_v, [pos], encs, mask=m)
            return n + jnp.sum(mi)

        n = lax.fori_loop(0, (cnt + L - 1) // L, grp_chunk, jnp.int32(0))

        def chunk_body(c, prev):
            for j in range(L):
                @pl.when(j < prev)
                def _():
                    _drain_one(ph_v, uall_hbm, semW)
            encs = blist_v[pl.ds(c * L, L)]
            bvec = encs & BMASK
            cvec = jnp.clip(
                lax.shift_right_logical(encs, BSHIFT) - col_base,
                0, WINC - 1)
            for d in range(DIM):
                vec = plsc.load_gather(
                    blk_v, [jnp.broadcast_to(buf, (L,)),
                            jnp.full((L,), d, jnp.int32), cvec])
                st_v[d, pl.ds(0, L)] = vec
            nrem = n - c * L
            cb = c & 1
            for j in range(L):
                @pl.when(j < nrem)
                def _():
                    b_s = bvec[j]
                    for k in range(DIM // L):
                        vk = plsc.load_gather(
                            st_v, [k * L + iota, jnp.full((L,), j, jnp.int32)])
                        ph_v[cb, j, pl.ds(k * L, L)] = vk
                    pltpu.async_copy(
                        ph_v.at[cb, j], uall_hbm.at[pl.ds(b_s * DIM, DIM)], semW)
            return jnp.minimum(nrem, L)

        last = lax.fori_loop(0, (n + L - 1) // L, chunk_body, jnp.int32(0))
        for j in range(L):
            @pl.when(j < last)
            def _():
                _drain_one(ph_v, uall_hbm, semW)

    pltpu.async_copy(
        tab_hbm.at[:, pl.ds(pl.multiple_of(wlo, 128), WINC)], blk_v.at[0], semB)

    def win_body(s, win_carry):
        buf = s & 1

        @pl.when(s + 1 < nwin)
        def _():
            pltpu.async_copy(
                tab_hbm.at[:, pl.ds(pl.multiple_of(wlo + (s + 1) * WINC, 128), WINC)],
                blk_v.at[(s + 1) & 1], semB)

        pltpu.make_async_copy(
            tab_hbm.at[:, pl.ds(0, WINC)], blk_v.at[buf], semB).wait()
        proc_block(buf, s * WINC, (s + 1) * WINC, s * WINC)
        return win_carry

    lax.fori_loop(0, nwin, win_body, 0)


def _scan_body(uids_hbm, iids_hbm, utab_hbm, itab_hbm,
               uall_hbm, vall_hbm,
               ids_v, slist_v, blist_v, blk_v, st_v, ph_v, semB, semW):
    wid = lax.axis_index("s") * NC + lax.axis_index("c")
    args = (wid, ids_v, slist_v, blist_v, blk_v, st_v, ph_v, semB, semW)
    _pass(uids_hbm, utab_hbm, uall_hbm, *args)
    _pass(iids_hbm, itab_hbm, vall_hbm, *args)


def _dot_body(uall_hbm, vall_hbm, uids_hbm, iids_hbm, utail_hbm, itail_hbm,
              out_hbm, ubuf_v, ibuf_v, utail_v, itail_v, out_v,
              uid_v, iid_v, sem):
    wid = lax.axis_index("s") * NC + lax.axis_index("c")
    base = wid * BPW
    cu = pltpu.async_copy(uall_hbm.at[pl.ds(base * DIM, BPW * DIM)], ubuf_v, sem)
    ci = pltpu.async_copy(vall_hbm.at[pl.ds(base * DIM, BPW * DIM)], ibuf_v, sem)
    pltpu.sync_copy(uids_hbm.at[pl.ds(base, BPW)], uid_v)
    pltpu.sync_copy(iids_hbm.at[pl.ds(base, BPW)], iid_v)
    pltpu.sync_copy(utail_hbm, utail_v)
    pltpu.sync_copy(itail_hbm, itail_v)
    cu.wait()
    ci.wait()
    rows_iota = lax.iota(jnp.int32, L)

    def patch(c, carry, id_v, tail_v, buf_v):
        idv = id_v[pl.ds(c * L, L)]
        for r in range(L):
            id_s = idv[r]

            @pl.when(id_s >= SPECIAL_R0)
            def _():
                col = jnp.broadcast_to(id_s - SPECIAL_R0, (L,))
                for k in range(DIM // L):
                    vk = plsc.load_gather(tail_v, [k * L + rows_iota, col])
                    buf_v[pl.ds((c * L + r) * DIM + k * L, L)] = vk
        return carry

    lax.fori_loop(0, BPW // L, functools.partial(
        patch, id_v=uid_v, tail_v=utail_v, buf_v=ubuf_v), 0)
    lax.fori_loop(0, BPW // L, functools.partial(
        patch, id_v=iid_v, tail_v=itail_v, buf_v=ibuf_v), 0)

    def group(g, carry):
        outvec = jnp.zeros((L,), jnp.float32)
        for r in range(L):
            row = g * L + r
            acc = ubuf_v[pl.ds(row * DIM, L)] * ibuf_v[pl.ds(row * DIM, L)]
            for k in range(1, DIM // L):
                acc = acc + (ubuf_v[pl.ds(row * DIM + k * L, L)]
                             * ibuf_v[pl.ds(row * DIM + k * L, L)])
            outvec = jnp.where(rows_iota == r, jnp.sum(acc), outvec)
        out_v[pl.ds(g * L, L)] = outvec
        return carry

    lax.fori_loop(0, BPW // L, group, 0)
    pltpu.sync_copy(out_v, out_hbm.at[pl.ds(base, BPW)])


def kernel(user_ids, item_ids, user_table, item_table):
    utab_t = user_table.T
    itab_t = item_table.T
    mesh = plsc.VectorSubcoreMesh(core_axis_name="c", subcore_axis_name="s")
    params = pltpu.CompilerParams(needs_layout_passes=False)

    scan = functools.partial(
        pl.kernel,
        out_type=(jax.ShapeDtypeStruct((BATCH * DIM,), jnp.float32),
                  jax.ShapeDtypeStruct((BATCH * DIM,), jnp.float32)),
        mesh=mesh,
        compiler_params=params,
        scratch_types=[
            pltpu.VMEM((4096,), jnp.int32),
            pltpu.VMEM((BATCH,), jnp.int32),
            pltpu.VMEM((BATCH,), jnp.int32),
            pltpu.VMEM((2, DIM, WINC), jnp.float32),
            pltpu.VMEM((DIM, L + 1), jnp.float32),
            pltpu.VMEM((2, L, DIM), jnp.float32),
            pltpu.SemaphoreType.DMA,
            pltpu.SemaphoreType.DMA,
        ],
    )(_scan_body)
    u_all, v_all = scan(user_ids.astype(jnp.int32), item_ids.astype(jnp.int32),
                        utab_t, itab_t)

    utail_t = user_table[SPECIAL_R0:].T
    itail_t = item_table[SPECIAL_R0:].T
    dot = functools.partial(
        pl.kernel,
        out_type=jax.ShapeDtypeStruct((BATCH,), jnp.float32),
        mesh=mesh,
        compiler_params=params,
        scratch_types=[
            pltpu.VMEM((BPW * DIM,), jnp.float32),
            pltpu.VMEM((BPW * DIM,), jnp.float32),
            pltpu.VMEM((DIM, SPECIAL_W), jnp.float32),
            pltpu.VMEM((DIM, SPECIAL_W), jnp.float32),
            pltpu.VMEM((BPW,), jnp.float32),
            pltpu.VMEM((BPW,), jnp.int32),
            pltpu.VMEM((BPW,), jnp.int32),
            pltpu.SemaphoreType.DMA,
        ],
    )(_dot_body)
    return dot(u_all, v_all, user_ids.astype(jnp.int32),
               item_ids.astype(jnp.int32), utail_t, itail_t)
